# baseline (speedup 1.0000x reference)
import functools

import jax
import jax.numpy as jnp
from jax import lax
from jax.experimental import pallas as pl
from jax.experimental.pallas import tpu as pltpu
from jax.experimental.pallas import tpu_sc as plsc

NC = 2
NS = 16
L = 16
CH = 128


@functools.lru_cache(maxsize=None)
def _build(B: int, D: int):
    NW = NC * NS
    b_per_w = B // NW
    n_ch = b_per_w // CH
    n_grp = b_per_w // L

    mesh = plsc.VectorSubcoreMesh(
        core_axis_name="c", subcore_axis_name="s",
        num_cores=NC, num_subcores=NS)

    @functools.partial(
        pl.kernel,
        out_type=jax.ShapeDtypeStruct((B,), jnp.float32),
        mesh=mesh,
        scratch_types=[
            pltpu.VMEM((n_ch, CH), jnp.int32),
            pltpu.VMEM((n_ch, CH), jnp.int32),
            pltpu.VMEM((b_per_w, D), jnp.float32),
            pltpu.VMEM((b_per_w, D), jnp.float32),
            pltpu.VMEM((b_per_w,), jnp.float32),
            pltpu.SemaphoreType.DMA,
        ],
        compiler_params=pltpu.CompilerParams(
            needs_layout_passes=False, use_tc_tiling_on_sc=False),
    )
    def dis_kernel(src_hbm, dst_hbm, table_hbm, out_hbm,
                   sidx, didx, srows, drows, obuf, sem):
        wid = lax.axis_index("s") * NC + lax.axis_index("c")
        base_ch = wid * n_ch

        pltpu.sync_copy(src_hbm.at[pl.ds(base_ch, n_ch)], sidx)
        pltpu.sync_copy(dst_hbm.at[pl.ds(base_ch, n_ch)], didx)

        copies = []
        for c in range(n_ch):
            copies.append(pltpu.async_copy(
                table_hbm.at[sidx.at[c]],
                srows.at[pl.ds(c * CH, CH)], sem))
            copies.append(pltpu.async_copy(
                table_hbm.at[didx.at[c]],
                drows.at[pl.ds(c * CH, CH)], sem))
        for cp in copies:
            cp.wait()

        lane_iota = lax.iota(jnp.int32, L)

        def group(g, carry):
            lanes = g * L + lane_iota
            acc = jnp.zeros((L,), jnp.float32)
            for d in range(D):
                col = jnp.full((L,), d, jnp.int32)
                s = plsc.load_gather(srows, [lanes, col])
                t = plsc.load_gather(drows, [lanes, col])
                df = s - t
                acc = acc + df * df
            x = acc + jnp.float32(1e-12)
            i = plsc.bitcast(x, jnp.int32)
            r = plsc.bitcast(jnp.int32(0x5F3759DF) - (i >> 1), jnp.float32)
            half_x = jnp.float32(0.5) * x
            for _ in range(3):
                r = r * (jnp.float32(1.5) - half_x * r * r)
            obuf[pl.ds(g * L, L)] = x * r
            return carry

        lax.fori_loop(0, n_grp, group, 0)
        pltpu.sync_copy(obuf, out_hbm.at[pl.ds(wid * b_per_w, b_per_w)])

    return dis_kernel


def kernel(input_triplet, table):
    B = input_triplet.shape[0]
    D = table.shape[1]
    src = input_triplet[:, 0].astype(jnp.int32).reshape(B // CH, CH)
    dst = input_triplet[:, 1].astype(jnp.int32).reshape(B // CH, CH)
    return _build(B, D)(src, dst, table)
